# Initial kernel scaffold; baseline (speedup 1.0000x reference)
import jax
import jax.numpy as jnp
from jax.experimental import pallas as pl


def kernel(x, edge_index, edge_latent, y, params):
    raise NotImplementedError("write your pallas kernel here")


# speedup vs baseline: 1.0029x; 1.0029x over previous
import jax
import jax.numpy as jnp
from jax.experimental import pallas as pl
from jax.experimental.pallas import tpu as pltpu

N = 10000
E = 320000
B = 20
FEAT = 512
HID = 128
BUF = 512
K = 16
NC = 4
ROWS = N // B


def _layernorm(x, g, b):
    mu = jnp.mean(x, axis=-1, keepdims=True)
    var = jnp.var(x, axis=-1, keepdims=True)
    return (x - mu) / jnp.sqrt(var + 1e-5) * g + b


def _genconv(x, src, dst, p):
    msg = jax.nn.relu(x[src]) + 1e-7
    logits = msg * p['t']
    m = jax.ops.segment_max(logits, dst, num_segments=x.shape[0])
    m = jnp.where(jnp.isfinite(m), m, 0.0)
    ex = jnp.exp(logits - m[dst])
    den = jax.ops.segment_sum(ex, dst, num_segments=x.shape[0])
    alpha = ex / (den[dst] + 1e-16)
    aggr = jax.ops.segment_sum(alpha * msg, dst, num_segments=x.shape[0])
    h = x + aggr
    h = h @ p['W1'] + p['b1']
    h = _layernorm(h, p['lg'], p['lb'])
    h = jax.nn.relu(h)
    return h @ p['W2'] + p['b2']


def _attn_body(x_ref, wphi_ref, bphi_ref, wa_ref, ba_ref, wb_ref, bb_ref,
               wc_ref, bc_ref, wrho_ref, brho_ref, wcls_ref, bcls_ref,
               logits_ref, a_path_ref, hfin_ref):
    xb = x_ref[0]
    hp = jnp.maximum(
        jnp.dot(xb, wphi_ref[...], preferred_element_type=jnp.float32)
        + bphi_ref[...], 0.0)
    a = jnp.tanh(
        jnp.dot(hp, wa_ref[...], preferred_element_type=jnp.float32)
        + ba_ref[...])
    bg = jax.nn.sigmoid(
        jnp.dot(hp, wb_ref[...], preferred_element_type=jnp.float32)
        + bb_ref[...])
    gated = a * bg
    A = jnp.sum(gated * wc_ref[...], axis=1, keepdims=True) + bc_ref[...]
    a_path_ref[0, 0, :] = A[:, 0]
    m = jnp.max(A, axis=0, keepdims=True)
    e = jnp.exp(A - m)
    sm = e / jnp.sum(e, axis=0, keepdims=True)
    pooled = jnp.dot(sm.T, hp, preferred_element_type=jnp.float32)
    hf = jnp.maximum(
        jnp.dot(pooled, wrho_ref[...], preferred_element_type=jnp.float32)
        + brho_ref[...], 0.0)
    hfin_ref[0] = hf
    logits_ref[0] = (
        jnp.dot(hf, wcls_ref[...], preferred_element_type=jnp.float32)
        + bcls_ref[...])


def _attn_pool(x_, p):
    row = lambda v: v.reshape(1, -1)
    w_spec = lambda s: pl.BlockSpec(s, lambda b: (0,) * len(s))
    out_shapes = (
        jax.ShapeDtypeStruct((B, 1, NC), jnp.float32),
        jax.ShapeDtypeStruct((B, 1, ROWS), jnp.float32),
        jax.ShapeDtypeStruct((B, 1, FEAT), jnp.float32),
    )
    return pl.pallas_call(
        _attn_body,
        grid=(B,),
        out_shape=out_shapes,
        in_specs=[
            pl.BlockSpec((1, ROWS, FEAT), lambda b: (b, 0, 0)),
            w_spec((FEAT, FEAT)), w_spec((1, FEAT)),
            w_spec((FEAT, FEAT)), w_spec((1, FEAT)),
            w_spec((FEAT, FEAT)), w_spec((1, FEAT)),
            w_spec((1, FEAT)), w_spec((1, 1)),
            w_spec((FEAT, FEAT)), w_spec((1, FEAT)),
            w_spec((FEAT, NC)), w_spec((1, NC)),
        ],
        out_specs=(
            pl.BlockSpec((1, 1, NC), lambda b: (b, 0, 0)),
            pl.BlockSpec((1, 1, ROWS), lambda b: (b, 0, 0)),
            pl.BlockSpec((1, 1, FEAT), lambda b: (b, 0, 0)),
        ),
    )(
        x_.reshape(B, ROWS, FEAT),
        p['Wphi'], row(p['bphi']),
        p['Wa'], row(p['ba']),
        p['Wb'], row(p['bb']),
        p['Wc'].reshape(1, FEAT), p['bc'].reshape(1, 1),
        p['Wrho'], row(p['brho']),
        p['Wcls'], row(p['bcls']),
    )


def kernel(x, edge_index, edge_latent, y, params):
    p = params
    src = edge_index[0]
    dst = edge_index[1]
    h = jax.nn.relu(x @ p['Wfc'] + p['bfc'])
    x_ = h
    cur = _genconv(h, src, dst, p['gens'][0])
    x_ = jnp.concatenate([x_, cur], axis=-1)
    for g in p['gens'][1:]:
        t = _layernorm(cur, g['ng'], g['nb'])
        t = jax.nn.relu(t)
        t = _genconv(t, src, dst, g)
        cur = cur + t
        x_ = jnp.concatenate([x_, cur], axis=-1)

    logits3, A_path, h_fin3 = _attn_pool(x_, p)
    logits = logits3.reshape(B, NC)
    h_fin = h_fin3.reshape(B, FEAT)

    x_concat = jnp.concatenate([h_fin, p['rehearsal']], axis=0)[:BUF]
    d2 = jnp.sum(x_concat ** 2, axis=1)
    dist = d2[:, None] + d2[None, :] - 2.0 * (x_concat @ x_concat.T)
    simk, idx = jax.lax.top_k(-dist, K)
    edge_attr = jax.nn.softmax(simk, axis=-1).reshape(-1)
    gsrc = idx.reshape(-1)
    gdst = jnp.repeat(jnp.arange(BUF), K)
    agg = jax.ops.segment_sum(edge_attr[:, None] * x_concat[gsrc], gdst, num_segments=BUF)
    hg = jax.nn.relu(agg @ p['Wg1'] + p['bg1'])
    agg2 = jax.ops.segment_sum(edge_attr[:, None] * hg[gsrc], gdst, num_segments=BUF)
    logits_graph = (agg2 @ p['Wg2'] + p['bg2'])[:B]
    return logits, A_path, logits_graph


# speedup vs baseline: 3.6534x; 3.6429x over previous
import functools

import jax
import jax.numpy as jnp
from jax import lax
from jax.experimental import pallas as pl
from jax.experimental.pallas import tpu as pltpu
from jax.experimental.pallas import tpu_sc as plsc

N = 10000
E = 320000
B = 20
FEAT = 512
HID = 128
BUF = 512
K = 16
NC = 4
ROWS = N // B

HALF = HID // 2
NSUB = 16
G = 64
NB = 316
NB2 = NB // 2
EPT = NB * G
EPAD = EPT * NSUB
NPAD = 10240
RPT = NPAD // NSUB


def _edge_compute(gbuf, cbuf, tv):
    def row(r, carry):
        for jj in range(HALF // 16):
            m = gbuf[r, pl.ds(16 * jj, 16)]
            e = jnp.exp(m * tv)
            cbuf[r, pl.ds(16 * jj, 16)] = e * m
            cbuf[r, pl.ds(HALF + 16 * jj, 16)] = e
        return carry
    lax.fori_loop(0, G, row, 0)


def _genconv_sc_body(u2_hbm, pk_hbm, zeros_hbm, t_hbm, out_hbm,
                     idxbuf0, idxbuf1, gbuf0, gbuf1, cbuf0, cbuf1,
                     dstbuf0, dstbuf1, tvmem, acc,
                     isem0, isem1, gsem0, gsem1, ssem0, ssem1):
    cid = lax.axis_index("c")
    sid = lax.axis_index("s")
    pltpu.sync_copy(t_hbm, tvmem)
    pltpu.sync_copy(zeros_hbm, acc.at[pl.ds(sid * RPT, RPT)])
    plsc.subcore_barrier()
    tv = tvmem[...]
    pkbase = sid * (2 * EPT)

    def idx_fetch(j, idxbuf, isem):
        pltpu.async_copy(pk_hbm.at[cid, pl.ds(pkbase + j * 2 * G, 2 * G)],
                         idxbuf, isem)

    def idx_wait(idxbuf, isem):
        pltpu.make_async_copy(pk_hbm.at[cid, pl.ds(0, 2 * G)], idxbuf,
                              isem).wait()

    def gather(idxbuf, gbuf, gsem):
        pltpu.async_copy(u2_hbm.at[idxbuf.at[pl.ds(0, G)]], gbuf, gsem)

    def gwait(idxbuf, gbuf, gsem):
        pltpu.make_async_copy(u2_hbm.at[idxbuf.at[pl.ds(0, G)]], gbuf,
                              gsem).wait()

    def swait(cbuf, dstbuf, ssem):
        pltpu.make_async_copy(cbuf, acc.at[dstbuf], ssem).wait()

    idx_fetch(0, idxbuf0, isem0)
    idx_fetch(1, idxbuf1, isem1)
    idx_wait(idxbuf0, isem0)
    gather(idxbuf0, gbuf0, gsem0)
    idx_wait(idxbuf1, isem1)
    gather(idxbuf1, gbuf1, gsem1)

    def slot(i, j, idxbuf, gbuf, cbuf, dstbuf, isem, gsem, ssem):
        gwait(idxbuf, gbuf, gsem)

        @pl.when(i > 0)
        def _():
            swait(cbuf, dstbuf, ssem)

        for ii in range(G // 16):
            dstbuf[pl.ds(16 * ii, 16)] = idxbuf[pl.ds(G + 16 * ii, 16)]

        @pl.when(j + 2 < NB)
        def _():
            idx_fetch(j + 2, idxbuf, isem)

        _edge_compute(gbuf, cbuf, tv)
        pltpu.async_copy(cbuf, acc.at[dstbuf], ssem, add=True)

        @pl.when(j + 2 < NB)
        def _():
            idx_wait(idxbuf, isem)
            gather(idxbuf, gbuf, gsem)

    def body(i, carry):
        slot(i, 2 * i, idxbuf0, gbuf0, cbuf0, dstbuf0, isem0, gsem0, ssem0)
        slot(i, 2 * i + 1, idxbuf1, gbuf1, cbuf1, dstbuf1, isem1, gsem1, ssem1)
        return carry

    lax.fori_loop(0, NB2, body, 0)
    swait(cbuf0, dstbuf0, ssem0)
    swait(cbuf1, dstbuf1, ssem1)
    plsc.subcore_barrier()

    for k in range(RPT // G):
        row0 = sid * RPT + k * G
        pltpu.sync_copy(acc.at[pl.ds(row0, G)], cbuf0)

        def drow(r, carry):
            for jj in range(HALF // 16):
                num = cbuf0[r, pl.ds(16 * jj, 16)]
                den = cbuf0[r, pl.ds(HALF + 16 * jj, 16)]
                gbuf0[r, pl.ds(16 * jj, 16)] = num / (den + 1e-16)
            return carry

        lax.fori_loop(0, G, drow, 0)
        pltpu.sync_copy(gbuf0, out_hbm.at[cid, pl.ds(row0, G)])


_genconv_sc = functools.partial(
    pl.kernel,
    _genconv_sc_body,
    out_type=jax.ShapeDtypeStruct((2, NPAD, HALF), jnp.float32),
    mesh=plsc.VectorSubcoreMesh(core_axis_name="c", subcore_axis_name="s"),
    compiler_params=pltpu.CompilerParams(use_tc_tiling_on_sc=False),
    scratch_types=[
        pltpu.VMEM((2 * G,), jnp.int32),
        pltpu.VMEM((2 * G,), jnp.int32),
        pltpu.VMEM((G, HALF), jnp.float32),
        pltpu.VMEM((G, HALF), jnp.float32),
        pltpu.VMEM((G, HID), jnp.float32),
        pltpu.VMEM((G, HID), jnp.float32),
        pltpu.VMEM((G,), jnp.int32),
        pltpu.VMEM((G,), jnp.int32),
        pltpu.VMEM((16,), jnp.float32),
        pltpu.VMEM_SHARED((NPAD, HID), jnp.float32),
        pltpu.SemaphoreType.DMA,
        pltpu.SemaphoreType.DMA,
        pltpu.SemaphoreType.DMA,
        pltpu.SemaphoreType.DMA,
        pltpu.SemaphoreType.DMA,
        pltpu.SemaphoreType.DMA,
    ],
)()


def _softmax_aggregate(v, pk, zeros, t):
    u = v + 1e-7
    u2 = jnp.concatenate([u[:, :HALF], u[:, HALF:]], axis=0)
    tvec = jnp.full((16,), t, jnp.float32)
    out = _genconv_sc(u2, pk, zeros, tvec)
    return jnp.concatenate([out[0, :N], out[1, :N]], axis=1)


def _pack_edges(src, dst):
    pad = EPAD - E
    src_pad = jnp.concatenate([src, jnp.zeros((pad,), src.dtype)])
    dst_pad = jnp.concatenate([dst, jnp.full((pad,), N, dst.dtype)])
    d = dst_pad.reshape(-1, G)
    pks = [
        jnp.concatenate([(src_pad + c * N).reshape(-1, G), d],
                        axis=1).reshape(-1)
        for c in range(2)
    ]
    return jnp.stack(pks)


def _layernorm(x, g, b):
    mu = jnp.mean(x, axis=-1, keepdims=True)
    var = jnp.var(x, axis=-1, keepdims=True)
    return (x - mu) / jnp.sqrt(var + 1e-5) * g + b


def _genconv_mlp(x, aggr, p):
    h = x + aggr
    h = h @ p['W1'] + p['b1']
    h = _layernorm(h, p['lg'], p['lb'])
    h = jax.nn.relu(h)
    return h @ p['W2'] + p['b2']


def _attn_body(x_ref, wphi_ref, bphi_ref, wa_ref, ba_ref, wb_ref, bb_ref,
               wc_ref, bc_ref, wrho_ref, brho_ref, wcls_ref, bcls_ref,
               logits_ref, a_path_ref, hfin_ref):
    xb = x_ref[0]
    hp = jnp.maximum(
        jnp.dot(xb, wphi_ref[...], preferred_element_type=jnp.float32)
        + bphi_ref[...], 0.0)
    a = jnp.tanh(
        jnp.dot(hp, wa_ref[...], preferred_element_type=jnp.float32)
        + ba_ref[...])
    bg = jax.nn.sigmoid(
        jnp.dot(hp, wb_ref[...], preferred_element_type=jnp.float32)
        + bb_ref[...])
    gated = a * bg
    A = jnp.sum(gated * wc_ref[...], axis=1, keepdims=True) + bc_ref[...]
    a_path_ref[0, 0, :] = A[:, 0]
    m = jnp.max(A, axis=0, keepdims=True)
    e = jnp.exp(A - m)
    sm = e / jnp.sum(e, axis=0, keepdims=True)
    pooled = jnp.dot(sm.T, hp, preferred_element_type=jnp.float32)
    hf = jnp.maximum(
        jnp.dot(pooled, wrho_ref[...], preferred_element_type=jnp.float32)
        + brho_ref[...], 0.0)
    hfin_ref[0] = hf
    logits_ref[0] = (
        jnp.dot(hf, wcls_ref[...], preferred_element_type=jnp.float32)
        + bcls_ref[...])


def _attn_pool(x_, p):
    row = lambda v: v.reshape(1, -1)
    w_spec = lambda s: pl.BlockSpec(s, lambda b: (0,) * len(s))
    out_shapes = (
        jax.ShapeDtypeStruct((B, 1, NC), jnp.float32),
        jax.ShapeDtypeStruct((B, 1, ROWS), jnp.float32),
        jax.ShapeDtypeStruct((B, 1, FEAT), jnp.float32),
    )
    return pl.pallas_call(
        _attn_body,
        grid=(B,),
        out_shape=out_shapes,
        in_specs=[
            pl.BlockSpec((1, ROWS, FEAT), lambda b: (b, 0, 0)),
            w_spec((FEAT, FEAT)), w_spec((1, FEAT)),
            w_spec((FEAT, FEAT)), w_spec((1, FEAT)),
            w_spec((FEAT, FEAT)), w_spec((1, FEAT)),
            w_spec((1, FEAT)), w_spec((1, 1)),
            w_spec((FEAT, FEAT)), w_spec((1, FEAT)),
            w_spec((FEAT, NC)), w_spec((1, NC)),
        ],
        out_specs=(
            pl.BlockSpec((1, 1, NC), lambda b: (b, 0, 0)),
            pl.BlockSpec((1, 1, ROWS), lambda b: (b, 0, 0)),
            pl.BlockSpec((1, 1, FEAT), lambda b: (b, 0, 0)),
        ),
    )(
        x_.reshape(B, ROWS, FEAT),
        p['Wphi'], row(p['bphi']),
        p['Wa'], row(p['ba']),
        p['Wb'], row(p['bb']),
        p['Wc'].reshape(1, FEAT), p['bc'].reshape(1, 1),
        p['Wrho'], row(p['brho']),
        p['Wcls'], row(p['bcls']),
    )


def kernel(x, edge_index, edge_latent, y, params):
    p = params
    pk = _pack_edges(edge_index[0], edge_index[1])
    zeros = jnp.zeros((RPT, HID), jnp.float32)

    h = jax.nn.relu(x @ p['Wfc'] + p['bfc'])
    x_ = h
    g0 = p['gens'][0]
    aggr = _softmax_aggregate(h, pk, zeros, g0['t'])
    cur = _genconv_mlp(h, aggr, g0)
    x_ = jnp.concatenate([x_, cur], axis=-1)
    for g in p['gens'][1:]:
        t = jax.nn.relu(_layernorm(cur, g['ng'], g['nb']))
        aggr = _softmax_aggregate(t, pk, zeros, g['t'])
        cur = cur + _genconv_mlp(t, aggr, g)
        x_ = jnp.concatenate([x_, cur], axis=-1)

    logits3, A_path, h_fin3 = _attn_pool(x_, p)
    logits = logits3.reshape(B, NC)
    h_fin = h_fin3.reshape(B, FEAT)

    x_concat = jnp.concatenate([h_fin, p['rehearsal']], axis=0)[:BUF]
    d2 = jnp.sum(x_concat ** 2, axis=1)
    dist = d2[:, None] + d2[None, :] - 2.0 * (x_concat @ x_concat.T)
    simk, idx = jax.lax.top_k(-dist, K)
    edge_attr = jax.nn.softmax(simk, axis=-1).reshape(-1)
    gsrc = idx.reshape(-1)
    gdst = jnp.repeat(jnp.arange(BUF), K)
    agg = jax.ops.segment_sum(edge_attr[:, None] * x_concat[gsrc], gdst, num_segments=BUF)
    hg = jax.nn.relu(agg @ p['Wg1'] + p['bg1'])
    agg2 = jax.ops.segment_sum(edge_attr[:, None] * hg[gsrc], gdst, num_segments=BUF)
    logits_graph = (agg2 @ p['Wg2'] + p['bg2'])[:B]
    return logits, A_path, logits_graph


# speedup vs baseline: 7.2989x; 1.9978x over previous
import functools

import jax
import jax.numpy as jnp
from jax import lax
from jax.experimental import pallas as pl
from jax.experimental.pallas import tpu as pltpu
from jax.experimental.pallas import tpu_sc as plsc

N = 10000
E = 320000
B = 20
FEAT = 512
HID = 128
BUF = 512
K = 16
NC = 4
ROWS = N // B

HALF = HID // 2
NSUB = 16
G = 64
NB = 316
NB2 = NB // 2
EPT = NB * G
EPAD = EPT * NSUB
NPAD = 10240
RPT = NPAD // NSUB


def _edge_compute(gbuf, cbuf, tv):
    nj = HALF // 16

    def row(r, carry):
        ms = [gbuf[r, pl.ds(16 * jj, 16)] for jj in range(nj)]
        es = [jnp.exp(m * tv) for m in ms]
        for jj in range(nj):
            cbuf[r, pl.ds(16 * jj, 16)] = es[jj] * ms[jj]
        for jj in range(nj):
            cbuf[r, pl.ds(HALF + 16 * jj, 16)] = es[jj]
        return carry

    lax.fori_loop(0, G, row, 0)


def _genconv_sc_body(u2_hbm, pk_hbm, zeros_hbm, t_hbm, out_hbm,
                     idxbuf0, idxbuf1, gbuf0, gbuf1, cbuf0, cbuf1,
                     dstbuf0, dstbuf1, tvmem, acc,
                     isem0, isem1, gsem0, gsem1, ssem0, ssem1):
    cid = lax.axis_index("c")
    sid = lax.axis_index("s")
    pltpu.sync_copy(t_hbm, tvmem)
    pltpu.sync_copy(zeros_hbm, acc.at[pl.ds(sid * RPT, RPT)])
    plsc.subcore_barrier()
    tv = tvmem[...]
    pkbase = sid * (2 * EPT)

    def idx_fetch(j, idxbuf, isem):
        pltpu.async_copy(pk_hbm.at[cid, pl.ds(pkbase + j * 2 * G, 2 * G)],
                         idxbuf, isem)

    def idx_wait(idxbuf, isem):
        pltpu.make_async_copy(pk_hbm.at[cid, pl.ds(0, 2 * G)], idxbuf,
                              isem).wait()

    def gather(idxbuf, gbuf, gsem):
        pltpu.async_copy(u2_hbm.at[idxbuf.at[pl.ds(0, G)]], gbuf, gsem)

    def gwait(idxbuf, gbuf, gsem):
        pltpu.make_async_copy(u2_hbm.at[idxbuf.at[pl.ds(0, G)]], gbuf,
                              gsem).wait()

    def swait(cbuf, dstbuf, ssem):
        pltpu.make_async_copy(cbuf, acc.at[dstbuf], ssem).wait()

    idx_fetch(0, idxbuf0, isem0)
    idx_fetch(1, idxbuf1, isem1)
    idx_wait(idxbuf0, isem0)
    gather(idxbuf0, gbuf0, gsem0)
    idx_wait(idxbuf1, isem1)
    gather(idxbuf1, gbuf1, gsem1)

    def slot(i, j, idxbuf, gbuf, cbuf, dstbuf, isem, gsem, ssem):
        gwait(idxbuf, gbuf, gsem)

        @pl.when(i > 0)
        def _():
            swait(cbuf, dstbuf, ssem)

        for ii in range(G // 16):
            dstbuf[pl.ds(16 * ii, 16)] = idxbuf[pl.ds(G + 16 * ii, 16)]

        @pl.when(j + 2 < NB)
        def _():
            idx_fetch(j + 2, idxbuf, isem)

        _edge_compute(gbuf, cbuf, tv)
        pltpu.async_copy(cbuf, acc.at[dstbuf], ssem, add=True)

        @pl.when(j + 2 < NB)
        def _():
            idx_wait(idxbuf, isem)
            gather(idxbuf, gbuf, gsem)

    def body(i, carry):
        slot(i, 2 * i, idxbuf0, gbuf0, cbuf0, dstbuf0, isem0, gsem0, ssem0)
        slot(i, 2 * i + 1, idxbuf1, gbuf1, cbuf1, dstbuf1, isem1, gsem1, ssem1)
        return carry

    lax.fori_loop(0, NB2, body, 0)
    swait(cbuf0, dstbuf0, ssem0)
    swait(cbuf1, dstbuf1, ssem1)
    plsc.subcore_barrier()

    for k in range(RPT // G):
        row0 = sid * RPT + k * G
        pltpu.sync_copy(acc.at[pl.ds(row0, G)], cbuf0)

        def drow(r, carry):
            for jj in range(HALF // 16):
                num = cbuf0[r, pl.ds(16 * jj, 16)]
                den = cbuf0[r, pl.ds(HALF + 16 * jj, 16)]
                gbuf0[r, pl.ds(16 * jj, 16)] = num / (den + 1e-16)
            return carry

        lax.fori_loop(0, G, drow, 0)
        pltpu.sync_copy(gbuf0, out_hbm.at[cid, pl.ds(row0, G)])


_genconv_sc = functools.partial(
    pl.kernel,
    _genconv_sc_body,
    out_type=jax.ShapeDtypeStruct((2, NPAD, HALF), jnp.float32),
    mesh=plsc.VectorSubcoreMesh(core_axis_name="c", subcore_axis_name="s"),
    compiler_params=pltpu.CompilerParams(use_tc_tiling_on_sc=False),
    scratch_types=[
        pltpu.VMEM((2 * G,), jnp.int32),
        pltpu.VMEM((2 * G,), jnp.int32),
        pltpu.VMEM((G, HALF), jnp.float32),
        pltpu.VMEM((G, HALF), jnp.float32),
        pltpu.VMEM((G, HID), jnp.float32),
        pltpu.VMEM((G, HID), jnp.float32),
        pltpu.VMEM((G,), jnp.int32),
        pltpu.VMEM((G,), jnp.int32),
        pltpu.VMEM((16,), jnp.float32),
        pltpu.VMEM_SHARED((NPAD, HID), jnp.float32),
        pltpu.SemaphoreType.DMA,
        pltpu.SemaphoreType.DMA,
        pltpu.SemaphoreType.DMA,
        pltpu.SemaphoreType.DMA,
        pltpu.SemaphoreType.DMA,
        pltpu.SemaphoreType.DMA,
    ],
)()


def _softmax_aggregate(v, pk, zeros, t):
    u = v + 1e-7
    u2 = jnp.concatenate([u[:, :HALF], u[:, HALF:]], axis=0)
    tvec = jnp.full((16,), t, jnp.float32)
    out = _genconv_sc(u2, pk, zeros, tvec)
    return jnp.concatenate([out[0, :N], out[1, :N]], axis=1)


def _pack_edges(src, dst):
    pad = EPAD - E
    src_pad = jnp.concatenate([src, jnp.zeros((pad,), src.dtype)])
    dst_pad = jnp.concatenate([dst, jnp.full((pad,), N, dst.dtype)])
    d = dst_pad.reshape(-1, G)
    pks = [
        jnp.concatenate([(src_pad + c * N).reshape(-1, G), d],
                        axis=1).reshape(-1)
        for c in range(2)
    ]
    return jnp.stack(pks)


def _layernorm(x, g, b):
    mu = jnp.mean(x, axis=-1, keepdims=True)
    var = jnp.var(x, axis=-1, keepdims=True)
    return (x - mu) / jnp.sqrt(var + 1e-5) * g + b


def _genconv_mlp(x, aggr, p):
    h = x + aggr
    h = h @ p['W1'] + p['b1']
    h = _layernorm(h, p['lg'], p['lb'])
    h = jax.nn.relu(h)
    return h @ p['W2'] + p['b2']


def _attn_body(x_ref, wphi_ref, bphi_ref, wa_ref, ba_ref, wb_ref, bb_ref,
               wc_ref, bc_ref, wrho_ref, brho_ref, wcls_ref, bcls_ref,
               logits_ref, a_path_ref, hfin_ref):
    xb = x_ref[0]
    hp = jnp.maximum(
        jnp.dot(xb, wphi_ref[...], preferred_element_type=jnp.float32)
        + bphi_ref[...], 0.0)
    a = jnp.tanh(
        jnp.dot(hp, wa_ref[...], preferred_element_type=jnp.float32)
        + ba_ref[...])
    bg = jax.nn.sigmoid(
        jnp.dot(hp, wb_ref[...], preferred_element_type=jnp.float32)
        + bb_ref[...])
    gated = a * bg
    A = jnp.sum(gated * wc_ref[...], axis=1, keepdims=True) + bc_ref[...]
    a_path_ref[0, 0, :] = A[:, 0]
    m = jnp.max(A, axis=0, keepdims=True)
    e = jnp.exp(A - m)
    sm = e / jnp.sum(e, axis=0, keepdims=True)
    pooled = jnp.dot(sm.T, hp, preferred_element_type=jnp.float32)
    hf = jnp.maximum(
        jnp.dot(pooled, wrho_ref[...], preferred_element_type=jnp.float32)
        + brho_ref[...], 0.0)
    hfin_ref[0] = hf
    logits_ref[0] = (
        jnp.dot(hf, wcls_ref[...], preferred_element_type=jnp.float32)
        + bcls_ref[...])


def _attn_pool(x_, p):
    row = lambda v: v.reshape(1, -1)
    w_spec = lambda s: pl.BlockSpec(s, lambda b: (0,) * len(s))
    out_shapes = (
        jax.ShapeDtypeStruct((B, 1, NC), jnp.float32),
        jax.ShapeDtypeStruct((B, 1, ROWS), jnp.float32),
        jax.ShapeDtypeStruct((B, 1, FEAT), jnp.float32),
    )
    return pl.pallas_call(
        _attn_body,
        grid=(B,),
        out_shape=out_shapes,
        in_specs=[
            pl.BlockSpec((1, ROWS, FEAT), lambda b: (b, 0, 0)),
            w_spec((FEAT, FEAT)), w_spec((1, FEAT)),
            w_spec((FEAT, FEAT)), w_spec((1, FEAT)),
            w_spec((FEAT, FEAT)), w_spec((1, FEAT)),
            w_spec((1, FEAT)), w_spec((1, 1)),
            w_spec((FEAT, FEAT)), w_spec((1, FEAT)),
            w_spec((FEAT, NC)), w_spec((1, NC)),
        ],
        out_specs=(
            pl.BlockSpec((1, 1, NC), lambda b: (b, 0, 0)),
            pl.BlockSpec((1, 1, ROWS), lambda b: (b, 0, 0)),
            pl.BlockSpec((1, 1, FEAT), lambda b: (b, 0, 0)),
        ),
    )(
        x_.reshape(B, ROWS, FEAT),
        p['Wphi'], row(p['bphi']),
        p['Wa'], row(p['ba']),
        p['Wb'], row(p['bb']),
        p['Wc'].reshape(1, FEAT), p['bc'].reshape(1, 1),
        p['Wrho'], row(p['brho']),
        p['Wcls'], row(p['bcls']),
    )


def kernel(x, edge_index, edge_latent, y, params):
    p = params
    pk = _pack_edges(edge_index[0], edge_index[1])
    zeros = jnp.zeros((RPT, HID), jnp.float32)

    h = jax.nn.relu(x @ p['Wfc'] + p['bfc'])
    x_ = h
    g0 = p['gens'][0]
    aggr = _softmax_aggregate(h, pk, zeros, g0['t'])
    cur = _genconv_mlp(h, aggr, g0)
    x_ = jnp.concatenate([x_, cur], axis=-1)
    for g in p['gens'][1:]:
        t = jax.nn.relu(_layernorm(cur, g['ng'], g['nb']))
        aggr = _softmax_aggregate(t, pk, zeros, g['t'])
        cur = cur + _genconv_mlp(t, aggr, g)
        x_ = jnp.concatenate([x_, cur], axis=-1)

    logits3, A_path, h_fin3 = _attn_pool(x_, p)
    logits = logits3.reshape(B, NC)
    h_fin = h_fin3.reshape(B, FEAT)

    x_concat = jnp.concatenate([h_fin, p['rehearsal']], axis=0)[:BUF]
    d2 = jnp.sum(x_concat ** 2, axis=1)
    dist = d2[:, None] + d2[None, :] - 2.0 * (x_concat @ x_concat.T)
    simk, idx = jax.lax.top_k(-dist, K)
    edge_attr = jax.nn.softmax(simk, axis=-1).reshape(-1)
    gsrc = idx.reshape(-1)
    gdst = jnp.repeat(jnp.arange(BUF), K)
    agg = jax.ops.segment_sum(edge_attr[:, None] * x_concat[gsrc], gdst, num_segments=BUF)
    hg = jax.nn.relu(agg @ p['Wg1'] + p['bg1'])
    agg2 = jax.ops.segment_sum(edge_attr[:, None] * hg[gsrc], gdst, num_segments=BUF)
    logits_graph = (agg2 @ p['Wg2'] + p['bg2'])[:B]
    return logits, A_path, logits_graph


# speedup vs baseline: 8.5819x; 1.1758x over previous
import functools

import jax
import jax.numpy as jnp
from jax import lax
from jax.experimental import pallas as pl
from jax.experimental.pallas import tpu as pltpu
from jax.experimental.pallas import tpu_sc as plsc

N = 10000
E = 320000
B = 20
FEAT = 512
HID = 128
BUF = 512
K = 16
NC = 4
ROWS = N // B

HALF = HID // 2
NSUB = 16
G = 64
NB = 316
NB2 = NB // 2
EPT = NB * G
EPAD = EPT * NSUB
NPAD = 10240
RPT = NPAD // NSUB


def _edge_compute(gbuf, cbuf, tv):
    nj = HALF // 16

    def row(r, carry):
        ms = [gbuf[r, pl.ds(16 * jj, 16)] for jj in range(nj)]
        es = [jnp.exp(m * tv) for m in ms]
        for jj in range(nj):
            cbuf[r, pl.ds(16 * jj, 16)] = es[jj] * ms[jj]
        for jj in range(nj):
            cbuf[r, pl.ds(HALF + 16 * jj, 16)] = es[jj]
        return carry

    lax.fori_loop(0, G, row, 0)


def _genconv_sc_body(u2_hbm, pk_hbm, zeros_hbm, t_hbm, out_hbm,
                     idxbuf0, idxbuf1, gbuf0, gbuf1, cbuf0, cbuf1,
                     dstbuf0, dstbuf1, tvmem, acc,
                     isem0, isem1, gsem0, gsem1, ssem0, ssem1):
    cid = lax.axis_index("c")
    sid = lax.axis_index("s")
    pltpu.sync_copy(t_hbm, tvmem)
    pltpu.sync_copy(zeros_hbm, acc.at[pl.ds(sid * RPT, RPT)])
    plsc.subcore_barrier()
    tv = tvmem[...]
    pkbase = sid * (2 * EPT)

    def idx_fetch(j, idxbuf, isem):
        pltpu.async_copy(pk_hbm.at[cid, pl.ds(pkbase + j * 2 * G, 2 * G)],
                         idxbuf, isem)

    def idx_wait(idxbuf, isem):
        pltpu.make_async_copy(pk_hbm.at[cid, pl.ds(0, 2 * G)], idxbuf,
                              isem).wait()

    def gather(idxbuf, gbuf, gsem):
        pltpu.async_copy(u2_hbm.at[idxbuf.at[pl.ds(0, G)]], gbuf, gsem)

    def gwait(idxbuf, gbuf, gsem):
        pltpu.make_async_copy(u2_hbm.at[idxbuf.at[pl.ds(0, G)]], gbuf,
                              gsem).wait()

    def swait(cbuf, dstbuf, ssem):
        pltpu.make_async_copy(cbuf, acc.at[dstbuf], ssem).wait()

    idx_fetch(0, idxbuf0, isem0)
    idx_fetch(1, idxbuf1, isem1)
    idx_wait(idxbuf0, isem0)
    gather(idxbuf0, gbuf0, gsem0)
    idx_wait(idxbuf1, isem1)
    gather(idxbuf1, gbuf1, gsem1)

    def slot(i, j, idxbuf, gbuf, cbuf, dstbuf, isem, gsem, ssem):
        gwait(idxbuf, gbuf, gsem)

        @pl.when(i > 0)
        def _():
            swait(cbuf, dstbuf, ssem)

        for ii in range(G // 16):
            dstbuf[pl.ds(16 * ii, 16)] = idxbuf[pl.ds(G + 16 * ii, 16)]

        @pl.when(j + 2 < NB)
        def _():
            idx_fetch(j + 2, idxbuf, isem)

        _edge_compute(gbuf, cbuf, tv)
        pltpu.async_copy(cbuf, acc.at[dstbuf], ssem, add=True)

        @pl.when(j + 2 < NB)
        def _():
            idx_wait(idxbuf, isem)
            gather(idxbuf, gbuf, gsem)

    def body(i, carry):
        slot(i, 2 * i, idxbuf0, gbuf0, cbuf0, dstbuf0, isem0, gsem0, ssem0)
        slot(i, 2 * i + 1, idxbuf1, gbuf1, cbuf1, dstbuf1, isem1, gsem1, ssem1)
        return carry

    lax.fori_loop(0, NB2, body, 0)
    swait(cbuf0, dstbuf0, ssem0)
    swait(cbuf1, dstbuf1, ssem1)
    plsc.subcore_barrier()

    for k in range(RPT // G):
        row0 = sid * RPT + k * G
        pltpu.sync_copy(acc.at[pl.ds(row0, G)], cbuf0)

        def drow(r, carry):
            for jj in range(HALF // 16):
                num = cbuf0[r, pl.ds(16 * jj, 16)]
                den = cbuf0[r, pl.ds(HALF + 16 * jj, 16)]
                gbuf0[r, pl.ds(16 * jj, 16)] = num / (den + 1e-16)
            return carry

        lax.fori_loop(0, G, drow, 0)
        pltpu.sync_copy(gbuf0, out_hbm.at[cid, pl.ds(row0, G)])


_genconv_sc = functools.partial(
    pl.kernel,
    _genconv_sc_body,
    out_type=jax.ShapeDtypeStruct((2, NPAD, HALF), jnp.float32),
    mesh=plsc.VectorSubcoreMesh(core_axis_name="c", subcore_axis_name="s"),
    compiler_params=pltpu.CompilerParams(use_tc_tiling_on_sc=False),
    scratch_types=[
        pltpu.VMEM((2 * G,), jnp.int32),
        pltpu.VMEM((2 * G,), jnp.int32),
        pltpu.VMEM((G, HALF), jnp.float32),
        pltpu.VMEM((G, HALF), jnp.float32),
        pltpu.VMEM((G, HID), jnp.float32),
        pltpu.VMEM((G, HID), jnp.float32),
        pltpu.VMEM((G,), jnp.int32),
        pltpu.VMEM((G,), jnp.int32),
        pltpu.VMEM((16,), jnp.float32),
        pltpu.VMEM_SHARED((NPAD, HID), jnp.float32),
        pltpu.SemaphoreType.DMA,
        pltpu.SemaphoreType.DMA,
        pltpu.SemaphoreType.DMA,
        pltpu.SemaphoreType.DMA,
        pltpu.SemaphoreType.DMA,
        pltpu.SemaphoreType.DMA,
    ],
)()


def _softmax_aggregate(v, pk, zeros, t):
    u = v + 1e-7
    u2 = jnp.concatenate([u[:, :HALF], u[:, HALF:]], axis=0)
    tvec = jnp.full((16,), t, jnp.float32)
    out = _genconv_sc(u2, pk, zeros, tvec)
    return jnp.concatenate([out[0, :N], out[1, :N]], axis=1)


def _pack_edges(src, dst):
    pad = EPAD - E
    src_pad = jnp.concatenate([src, jnp.zeros((pad,), src.dtype)])
    dst_pad = jnp.concatenate([dst, jnp.full((pad,), N, dst.dtype)])
    d = dst_pad.reshape(-1, G)
    pks = [
        jnp.concatenate([(src_pad + c * N).reshape(-1, G), d],
                        axis=1).reshape(-1)
        for c in range(2)
    ]
    return jnp.stack(pks)


def _layernorm(x, g, b):
    mu = jnp.mean(x, axis=-1, keepdims=True)
    var = jnp.var(x, axis=-1, keepdims=True)
    return (x - mu) / jnp.sqrt(var + 1e-5) * g + b


def _genconv_mlp(x, aggr, p):
    h = x + aggr
    h = h @ p['W1'] + p['b1']
    h = _layernorm(h, p['lg'], p['lb'])
    h = jax.nn.relu(h)
    return h @ p['W2'] + p['b2']


def _attn_body(x_ref, wphi_ref, bphi_ref, wa_ref, ba_ref, wb_ref, bb_ref,
               wc_ref, bc_ref, wrho_ref, brho_ref, wcls_ref, bcls_ref,
               logits_ref, a_path_ref, hfin_ref):
    xb = x_ref[0]
    hp = jnp.maximum(
        jnp.dot(xb, wphi_ref[...], preferred_element_type=jnp.float32)
        + bphi_ref[...], 0.0)
    a = jnp.tanh(
        jnp.dot(hp, wa_ref[...], preferred_element_type=jnp.float32)
        + ba_ref[...])
    bg = jax.nn.sigmoid(
        jnp.dot(hp, wb_ref[...], preferred_element_type=jnp.float32)
        + bb_ref[...])
    gated = a * bg
    A = jnp.sum(gated * wc_ref[...], axis=1, keepdims=True) + bc_ref[...]
    a_path_ref[0, 0, :] = A[:, 0]
    m = jnp.max(A, axis=0, keepdims=True)
    e = jnp.exp(A - m)
    sm = e / jnp.sum(e, axis=0, keepdims=True)
    pooled = jnp.dot(sm.T, hp, preferred_element_type=jnp.float32)
    hf = jnp.maximum(
        jnp.dot(pooled, wrho_ref[...], preferred_element_type=jnp.float32)
        + brho_ref[...], 0.0)
    hfin_ref[0] = hf
    logits_ref[0] = (
        jnp.dot(hf, wcls_ref[...], preferred_element_type=jnp.float32)
        + bcls_ref[...])


def _attn_pool(x_, p):
    row = lambda v: v.reshape(1, -1)
    w_spec = lambda s: pl.BlockSpec(s, lambda b: (0,) * len(s))
    out_shapes = (
        jax.ShapeDtypeStruct((B, 1, NC), jnp.float32),
        jax.ShapeDtypeStruct((B, 1, ROWS), jnp.float32),
        jax.ShapeDtypeStruct((B, 1, FEAT), jnp.float32),
    )
    return pl.pallas_call(
        _attn_body,
        grid=(B,),
        out_shape=out_shapes,
        in_specs=[
            pl.BlockSpec((1, ROWS, FEAT), lambda b: (b, 0, 0)),
            w_spec((FEAT, FEAT)), w_spec((1, FEAT)),
            w_spec((FEAT, FEAT)), w_spec((1, FEAT)),
            w_spec((FEAT, FEAT)), w_spec((1, FEAT)),
            w_spec((1, FEAT)), w_spec((1, 1)),
            w_spec((FEAT, FEAT)), w_spec((1, FEAT)),
            w_spec((FEAT, NC)), w_spec((1, NC)),
        ],
        out_specs=(
            pl.BlockSpec((1, 1, NC), lambda b: (b, 0, 0)),
            pl.BlockSpec((1, 1, ROWS), lambda b: (b, 0, 0)),
            pl.BlockSpec((1, 1, FEAT), lambda b: (b, 0, 0)),
        ),
    )(
        x_.reshape(B, ROWS, FEAT),
        p['Wphi'], row(p['bphi']),
        p['Wa'], row(p['ba']),
        p['Wb'], row(p['bb']),
        p['Wc'].reshape(1, FEAT), p['bc'].reshape(1, 1),
        p['Wrho'], row(p['brho']),
        p['Wcls'], row(p['bcls']),
    )


def _dsl_body(xc_ref, wg1_ref, bg1_ref, wg2_ref, bg2_ref, out_ref):
    xc = xc_ref[...]
    d2 = jnp.sum(xc * xc, axis=1, keepdims=True)
    nd = 2.0 * jnp.dot(xc, xc.T, preferred_element_type=jnp.float32) \
        - d2 - d2.T
    col = jax.lax.broadcasted_iota(jnp.int32, (BUF, BUF), 1)
    wd = jnp.zeros((BUF, BUF), jnp.float32)
    z = jnp.zeros((BUF, 1), jnp.float32)
    m0 = jnp.max(nd, axis=1, keepdims=True)
    for _ in range(K):
        m = jnp.max(nd, axis=1, keepdims=True)
        eq = nd >= m
        first = jnp.min(jnp.where(eq, col, BUF), axis=1, keepdims=True)
        oh = (col == first).astype(jnp.float32)
        e = jnp.exp(m - m0)
        wd = wd + e * oh
        z = z + e
        nd = jnp.where(oh > 0.0, -3.0e38, nd)
    zinv = 1.0 / z
    agg = jnp.dot(wd, xc, preferred_element_type=jnp.float32) * zinv
    hg = jnp.maximum(
        jnp.dot(agg, wg1_ref[...], preferred_element_type=jnp.float32)
        + bg1_ref[...], 0.0)
    agg2 = jnp.dot(wd, hg, preferred_element_type=jnp.float32) * zinv
    out_ref[...] = (
        jnp.dot(agg2, wg2_ref[...], preferred_element_type=jnp.float32)
        + bg2_ref[...])


def _dsl_graph(x_concat, p):
    return pl.pallas_call(
        _dsl_body,
        out_shape=jax.ShapeDtypeStruct((BUF, NC), jnp.float32),
    )(x_concat, p['Wg1'], p['bg1'].reshape(1, -1),
      p['Wg2'], p['bg2'].reshape(1, -1))


def kernel(x, edge_index, edge_latent, y, params):
    p = params
    pk = _pack_edges(edge_index[0], edge_index[1])
    zeros = jnp.zeros((RPT, HID), jnp.float32)

    h = jax.nn.relu(x @ p['Wfc'] + p['bfc'])
    x_ = h
    g0 = p['gens'][0]
    aggr = _softmax_aggregate(h, pk, zeros, g0['t'])
    cur = _genconv_mlp(h, aggr, g0)
    x_ = jnp.concatenate([x_, cur], axis=-1)
    for g in p['gens'][1:]:
        t = jax.nn.relu(_layernorm(cur, g['ng'], g['nb']))
        aggr = _softmax_aggregate(t, pk, zeros, g['t'])
        cur = cur + _genconv_mlp(t, aggr, g)
        x_ = jnp.concatenate([x_, cur], axis=-1)

    logits3, A_path, h_fin3 = _attn_pool(x_, p)
    logits = logits3.reshape(B, NC)
    h_fin = h_fin3.reshape(B, FEAT)

    x_concat = jnp.concatenate([h_fin, p['rehearsal']], axis=0)[:BUF]
    logits_graph = _dsl_graph(x_concat, p)[:B]
    return logits, A_path, logits_graph


# speedup vs baseline: 10.3075x; 1.2011x over previous
import functools

import jax
import jax.numpy as jnp
from jax import lax
from jax.experimental import pallas as pl
from jax.experimental.pallas import tpu as pltpu
from jax.experimental.pallas import tpu_sc as plsc

N = 10000
E = 320000
B = 20
FEAT = 512
HID = 128
BUF = 512
K = 16
NC = 4
ROWS = N // B

HALF = HID // 2
NSUB = 16
G = 64
NB = 316
NB2 = NB // 2
EPT = NB * G
EPAD = EPT * NSUB
NPAD = 10240
RPT = NPAD // NSUB


def _edge_compute(gbuf, cbuf, tv):
    nj = HALF // 16

    def row(r2, carry):
        r = 2 * r2
        ms = [(gbuf[r + k, pl.ds(16 * jj, 16)], r + k, jj)
              for k in range(2) for jj in range(nj)]
        es = [jnp.exp(m * tv) for m, _, _ in ms]
        for e, (m, rk, jj) in zip(es, ms):
            cbuf[rk, pl.ds(16 * jj, 16)] = e * m
        for e, (m, rk, jj) in zip(es, ms):
            cbuf[rk, pl.ds(HALF + 16 * jj, 16)] = e
        return carry

    lax.fori_loop(0, G // 2, row, 0)


def _genconv_sc_body(u2_hbm, pk_hbm, zeros_hbm, t_hbm, out_hbm,
                     idxbuf0, idxbuf1, gbuf0, gbuf1, cbuf0, cbuf1,
                     dstbuf0, dstbuf1, tvmem, acc,
                     isem0, isem1, gsem0, gsem1, ssem0, ssem1):
    cid = lax.axis_index("c")
    sid = lax.axis_index("s")
    pltpu.sync_copy(t_hbm, tvmem)
    pltpu.sync_copy(zeros_hbm, acc.at[pl.ds(sid * RPT, RPT)])
    plsc.subcore_barrier()
    tv = tvmem[...]
    pkbase = sid * (2 * EPT)

    def idx_fetch(j, idxbuf, isem):
        pltpu.async_copy(pk_hbm.at[cid, pl.ds(pkbase + j * 2 * G, 2 * G)],
                         idxbuf, isem)

    def idx_wait(idxbuf, isem):
        pltpu.make_async_copy(pk_hbm.at[cid, pl.ds(0, 2 * G)], idxbuf,
                              isem).wait()

    def gather(idxbuf, gbuf, gsem):
        pltpu.async_copy(u2_hbm.at[idxbuf.at[pl.ds(0, G)]], gbuf, gsem)

    def gwait(idxbuf, gbuf, gsem):
        pltpu.make_async_copy(u2_hbm.at[idxbuf.at[pl.ds(0, G)]], gbuf,
                              gsem).wait()

    def swait(cbuf, dstbuf, ssem):
        pltpu.make_async_copy(cbuf, acc.at[dstbuf], ssem).wait()

    idx_fetch(0, idxbuf0, isem0)
    idx_fetch(1, idxbuf1, isem1)
    idx_wait(idxbuf0, isem0)
    gather(idxbuf0, gbuf0, gsem0)
    idx_wait(idxbuf1, isem1)
    gather(idxbuf1, gbuf1, gsem1)

    def slot(i, j, idxbuf, gbuf, cbuf, dstbuf, isem, gsem, ssem):
        gwait(idxbuf, gbuf, gsem)

        @pl.when(i > 0)
        def _():
            swait(cbuf, dstbuf, ssem)

        for ii in range(G // 16):
            dstbuf[pl.ds(16 * ii, 16)] = idxbuf[pl.ds(G + 16 * ii, 16)]

        @pl.when(j + 2 < NB)
        def _():
            idx_fetch(j + 2, idxbuf, isem)

        _edge_compute(gbuf, cbuf, tv)
        pltpu.async_copy(cbuf, acc.at[dstbuf], ssem, add=True)

        @pl.when(j + 2 < NB)
        def _():
            idx_wait(idxbuf, isem)
            gather(idxbuf, gbuf, gsem)

    def body(i, carry):
        slot(i, 2 * i, idxbuf0, gbuf0, cbuf0, dstbuf0, isem0, gsem0, ssem0)
        slot(i, 2 * i + 1, idxbuf1, gbuf1, cbuf1, dstbuf1, isem1, gsem1, ssem1)
        return carry

    lax.fori_loop(0, NB2, body, 0)
    swait(cbuf0, dstbuf0, ssem0)
    swait(cbuf1, dstbuf1, ssem1)
    plsc.subcore_barrier()

    for k in range(RPT // G):
        row0 = sid * RPT + k * G
        pltpu.sync_copy(acc.at[pl.ds(row0, G)], cbuf0)

        def drow(r, carry):
            for jj in range(HALF // 16):
                num = cbuf0[r, pl.ds(16 * jj, 16)]
                den = cbuf0[r, pl.ds(HALF + 16 * jj, 16)]
                gbuf0[r, pl.ds(16 * jj, 16)] = num / (den + 1e-16)
            return carry

        lax.fori_loop(0, G, drow, 0)
        pltpu.sync_copy(gbuf0, out_hbm.at[cid, pl.ds(row0, G)])


_genconv_sc = functools.partial(
    pl.kernel,
    _genconv_sc_body,
    out_type=jax.ShapeDtypeStruct((2, NPAD, HALF), jnp.float32),
    mesh=plsc.VectorSubcoreMesh(core_axis_name="c", subcore_axis_name="s"),
    compiler_params=pltpu.CompilerParams(use_tc_tiling_on_sc=False),
    scratch_types=[
        pltpu.VMEM((2 * G,), jnp.int32),
        pltpu.VMEM((2 * G,), jnp.int32),
        pltpu.VMEM((G, HALF), jnp.float32),
        pltpu.VMEM((G, HALF), jnp.float32),
        pltpu.VMEM((G, HID), jnp.float32),
        pltpu.VMEM((G, HID), jnp.float32),
        pltpu.VMEM((G,), jnp.int32),
        pltpu.VMEM((G,), jnp.int32),
        pltpu.VMEM((16,), jnp.float32),
        pltpu.VMEM_SHARED((NPAD, HID), jnp.float32),
        pltpu.SemaphoreType.DMA,
        pltpu.SemaphoreType.DMA,
        pltpu.SemaphoreType.DMA,
        pltpu.SemaphoreType.DMA,
        pltpu.SemaphoreType.DMA,
        pltpu.SemaphoreType.DMA,
    ],
)()


def _softmax_aggregate(v, pk, zeros, t):
    u = v + 1e-7
    u2 = jnp.concatenate([u[:, :HALF], u[:, HALF:]], axis=0)
    tvec = jnp.full((16,), t, jnp.float32)
    out = _genconv_sc(u2, pk, zeros, tvec)
    return jnp.concatenate([out[0, :N], out[1, :N]], axis=1)


def _pack_edges(src, dst):
    pad = EPAD - E
    src_pad = jnp.concatenate([src, jnp.zeros((pad,), src.dtype)])
    dst_pad = jnp.concatenate([dst, jnp.full((pad,), N, dst.dtype)])
    d = dst_pad.reshape(-1, G)
    pks = [
        jnp.concatenate([(src_pad + c * N).reshape(-1, G), d],
                        axis=1).reshape(-1)
        for c in range(2)
    ]
    return jnp.stack(pks)


def _layernorm(x, g, b):
    mu = jnp.mean(x, axis=-1, keepdims=True)
    var = jnp.var(x, axis=-1, keepdims=True)
    return (x - mu) / jnp.sqrt(var + 1e-5) * g + b


def _genconv_mlp(x, aggr, p):
    h = x + aggr
    h = h @ p['W1'] + p['b1']
    h = _layernorm(h, p['lg'], p['lb'])
    h = jax.nn.relu(h)
    return h @ p['W2'] + p['b2']


def _attn_body(x_ref, wphi_ref, bphi_ref, wa_ref, ba_ref, wb_ref, bb_ref,
               wc_ref, bc_ref, wrho_ref, brho_ref, wcls_ref, bcls_ref,
               logits_ref, a_path_ref, hfin_ref):
    xb = x_ref[0]
    hp = jnp.maximum(
        jnp.dot(xb, wphi_ref[...], preferred_element_type=jnp.float32)
        + bphi_ref[...], 0.0)
    a = jnp.tanh(
        jnp.dot(hp, wa_ref[...], preferred_element_type=jnp.float32)
        + ba_ref[...])
    bg = jax.nn.sigmoid(
        jnp.dot(hp, wb_ref[...], preferred_element_type=jnp.float32)
        + bb_ref[...])
    gated = a * bg
    A = jnp.sum(gated * wc_ref[...], axis=1, keepdims=True) + bc_ref[...]
    a_path_ref[0, 0, :] = A[:, 0]
    m = jnp.max(A, axis=0, keepdims=True)
    e = jnp.exp(A - m)
    sm = e / jnp.sum(e, axis=0, keepdims=True)
    pooled = jnp.dot(sm.T, hp, preferred_element_type=jnp.float32)
    hf = jnp.maximum(
        jnp.dot(pooled, wrho_ref[...], preferred_element_type=jnp.float32)
        + brho_ref[...], 0.0)
    hfin_ref[0] = hf
    logits_ref[0] = (
        jnp.dot(hf, wcls_ref[...], preferred_element_type=jnp.float32)
        + bcls_ref[...])


def _attn_pool(x_, p):
    row = lambda v: v.reshape(1, -1)
    w_spec = lambda s: pl.BlockSpec(s, lambda b: (0,) * len(s))
    out_shapes = (
        jax.ShapeDtypeStruct((B, 1, NC), jnp.float32),
        jax.ShapeDtypeStruct((B, 1, ROWS), jnp.float32),
        jax.ShapeDtypeStruct((B, 1, FEAT), jnp.float32),
    )
    return pl.pallas_call(
        _attn_body,
        grid=(B,),
        out_shape=out_shapes,
        in_specs=[
            pl.BlockSpec((1, ROWS, FEAT), lambda b: (b, 0, 0)),
            w_spec((FEAT, FEAT)), w_spec((1, FEAT)),
            w_spec((FEAT, FEAT)), w_spec((1, FEAT)),
            w_spec((FEAT, FEAT)), w_spec((1, FEAT)),
            w_spec((1, FEAT)), w_spec((1, 1)),
            w_spec((FEAT, FEAT)), w_spec((1, FEAT)),
            w_spec((FEAT, NC)), w_spec((1, NC)),
        ],
        out_specs=(
            pl.BlockSpec((1, 1, NC), lambda b: (b, 0, 0)),
            pl.BlockSpec((1, 1, ROWS), lambda b: (b, 0, 0)),
            pl.BlockSpec((1, 1, FEAT), lambda b: (b, 0, 0)),
        ),
    )(
        x_.reshape(B, ROWS, FEAT),
        p['Wphi'], row(p['bphi']),
        p['Wa'], row(p['ba']),
        p['Wb'], row(p['bb']),
        p['Wc'].reshape(1, FEAT), p['bc'].reshape(1, 1),
        p['Wrho'], row(p['brho']),
        p['Wcls'], row(p['bcls']),
    )


def _dsl_body(xc_ref, wg1_ref, bg1_ref, wg2_ref, bg2_ref, out_ref):
    xc = xc_ref[...]
    d2 = jnp.sum(xc * xc, axis=1, keepdims=True)
    nd = 2.0 * jnp.dot(xc, xc.T, preferred_element_type=jnp.float32) \
        - d2 - d2.T
    col = jax.lax.broadcasted_iota(jnp.int32, (BUF, BUF), 1)
    wd = jnp.zeros((BUF, BUF), jnp.float32)
    z = jnp.zeros((BUF, 1), jnp.float32)
    m0 = jnp.max(nd, axis=1, keepdims=True)
    for _ in range(K):
        m = jnp.max(nd, axis=1, keepdims=True)
        eq = nd >= m
        first = jnp.min(jnp.where(eq, col, BUF), axis=1, keepdims=True)
        oh = (col == first).astype(jnp.float32)
        e = jnp.exp(m - m0)
        wd = wd + e * oh
        z = z + e
        nd = jnp.where(oh > 0.0, -3.0e38, nd)
    zinv = 1.0 / z
    agg = jnp.dot(wd, xc, preferred_element_type=jnp.float32) * zinv
    hg = jnp.maximum(
        jnp.dot(agg, wg1_ref[...], preferred_element_type=jnp.float32)
        + bg1_ref[...], 0.0)
    agg2 = jnp.dot(wd, hg, preferred_element_type=jnp.float32) * zinv
    out_ref[...] = (
        jnp.dot(agg2, wg2_ref[...], preferred_element_type=jnp.float32)
        + bg2_ref[...])


def _dsl_graph(x_concat, p):
    return pl.pallas_call(
        _dsl_body,
        out_shape=jax.ShapeDtypeStruct((BUF, NC), jnp.float32),
    )(x_concat, p['Wg1'], p['bg1'].reshape(1, -1),
      p['Wg2'], p['bg2'].reshape(1, -1))


def kernel(x, edge_index, edge_latent, y, params):
    p = params
    pk = _pack_edges(edge_index[0], edge_index[1])
    zeros = jnp.zeros((RPT, HID), jnp.float32)

    h = jax.nn.relu(x @ p['Wfc'] + p['bfc'])
    x_ = h
    g0 = p['gens'][0]
    aggr = _softmax_aggregate(h, pk, zeros, g0['t'])
    cur = _genconv_mlp(h, aggr, g0)
    x_ = jnp.concatenate([x_, cur], axis=-1)
    for g in p['gens'][1:]:
        t = jax.nn.relu(_layernorm(cur, g['ng'], g['nb']))
        aggr = _softmax_aggregate(t, pk, zeros, g['t'])
        cur = cur + _genconv_mlp(t, aggr, g)
        x_ = jnp.concatenate([x_, cur], axis=-1)

    logits3, A_path, h_fin3 = _attn_pool(x_, p)
    logits = logits3.reshape(B, NC)
    h_fin = h_fin3.reshape(B, FEAT)

    x_concat = jnp.concatenate([h_fin, p['rehearsal']], axis=0)[:BUF]
    logits_graph = _dsl_graph(x_concat, p)[:B]
    return logits, A_path, logits_graph


# speedup vs baseline: 10.3761x; 1.0067x over previous
import functools

import jax
import jax.numpy as jnp
from jax import lax
from jax.experimental import pallas as pl
from jax.experimental.pallas import tpu as pltpu
from jax.experimental.pallas import tpu_sc as plsc

N = 10000
E = 320000
B = 20
FEAT = 512
HID = 128
BUF = 512
K = 16
NC = 4
ROWS = N // B

HALF = HID // 2
NSUB = 16
G = 64
NB = 316
NB2 = NB // 2
EPT = NB * G
EPAD = EPT * NSUB
NPAD = 10240
RPT = NPAD // NSUB


def _edge_compute(gbuf, cbuf, tv):
    nj = HALF // 16

    def row(r2, carry):
        r = 2 * r2
        ms = [(gbuf[r + k, pl.ds(16 * jj, 16)], r + k, jj)
              for k in range(2) for jj in range(nj)]
        es = [jnp.exp(m * tv) for m, _, _ in ms]
        for e, (m, rk, jj) in zip(es, ms):
            cbuf[rk, pl.ds(16 * jj, 16)] = e * m
        for e, (m, rk, jj) in zip(es, ms):
            cbuf[rk, pl.ds(HALF + 16 * jj, 16)] = e
        return carry

    lax.fori_loop(0, G // 2, row, 0)


def _genconv_sc_body(u2_hbm, pk_hbm, zeros_hbm, t_hbm, out_hbm,
                     idxbuf0, idxbuf1, gbuf0, gbuf1, cbuf0, cbuf1,
                     dstbuf0, dstbuf1, tvmem, acc,
                     isem0, isem1, gsem0, gsem1, ssem0, ssem1):
    cid = lax.axis_index("c")
    sid = lax.axis_index("s")
    pltpu.sync_copy(t_hbm, tvmem)
    pltpu.sync_copy(zeros_hbm, acc.at[pl.ds(sid * RPT, RPT)])
    plsc.subcore_barrier()
    tv = tvmem[...]
    pkbase = sid * (2 * EPT)

    def idx_fetch(j, idxbuf, isem):
        pltpu.async_copy(pk_hbm.at[cid, pl.ds(pkbase + j * 2 * G, 2 * G)],
                         idxbuf, isem)

    def idx_wait(idxbuf, isem):
        pltpu.make_async_copy(pk_hbm.at[cid, pl.ds(0, 2 * G)], idxbuf,
                              isem).wait()

    def gather(idxbuf, gbuf, gsem):
        pltpu.async_copy(u2_hbm.at[idxbuf.at[pl.ds(0, G)]], gbuf, gsem)

    def gwait(idxbuf, gbuf, gsem):
        pltpu.make_async_copy(u2_hbm.at[idxbuf.at[pl.ds(0, G)]], gbuf,
                              gsem).wait()

    def swait(cbuf, dstbuf, ssem):
        pltpu.make_async_copy(cbuf, acc.at[dstbuf], ssem).wait()

    idx_fetch(0, idxbuf0, isem0)
    idx_fetch(1, idxbuf1, isem1)
    idx_wait(idxbuf0, isem0)
    gather(idxbuf0, gbuf0, gsem0)
    idx_wait(idxbuf1, isem1)
    gather(idxbuf1, gbuf1, gsem1)

    def slot(i, j, idxbuf, gbuf, cbuf, dstbuf, isem, gsem, ssem):
        gwait(idxbuf, gbuf, gsem)

        @pl.when(i > 0)
        def _():
            swait(cbuf, dstbuf, ssem)

        for ii in range(G // 16):
            dstbuf[pl.ds(16 * ii, 16)] = idxbuf[pl.ds(G + 16 * ii, 16)]

        @pl.when(j + 2 < NB)
        def _():
            idx_fetch(j + 2, idxbuf, isem)

        _edge_compute(gbuf, cbuf, tv)
        pltpu.async_copy(cbuf, acc.at[dstbuf], ssem, add=True)

        @pl.when(j + 2 < NB)
        def _():
            idx_wait(idxbuf, isem)
            gather(idxbuf, gbuf, gsem)

    def body(i, carry):
        slot(i, 2 * i, idxbuf0, gbuf0, cbuf0, dstbuf0, isem0, gsem0, ssem0)
        slot(i, 2 * i + 1, idxbuf1, gbuf1, cbuf1, dstbuf1, isem1, gsem1, ssem1)
        return carry

    lax.fori_loop(0, NB2, body, 0)
    swait(cbuf0, dstbuf0, ssem0)
    swait(cbuf1, dstbuf1, ssem1)
    plsc.subcore_barrier()

    for k in range(RPT // G):
        row0 = sid * RPT + k * G
        pltpu.sync_copy(acc.at[pl.ds(row0, G)], cbuf0)

        def drow(r, carry):
            for jj in range(HALF // 16):
                num = cbuf0[r, pl.ds(16 * jj, 16)]
                den = cbuf0[r, pl.ds(HALF + 16 * jj, 16)]
                gbuf0[r, pl.ds(16 * jj, 16)] = num / (den + 1e-16)
            return carry

        lax.fori_loop(0, G, drow, 0)
        pltpu.sync_copy(gbuf0, out_hbm.at[cid, pl.ds(row0, G)])


_genconv_sc = functools.partial(
    pl.kernel,
    _genconv_sc_body,
    out_type=jax.ShapeDtypeStruct((2, NPAD, HALF), jnp.float32),
    mesh=plsc.VectorSubcoreMesh(core_axis_name="c", subcore_axis_name="s"),
    compiler_params=pltpu.CompilerParams(use_tc_tiling_on_sc=False),
    scratch_types=[
        pltpu.VMEM((2 * G,), jnp.int32),
        pltpu.VMEM((2 * G,), jnp.int32),
        pltpu.VMEM((G, HALF), jnp.float32),
        pltpu.VMEM((G, HALF), jnp.float32),
        pltpu.VMEM((G, HID), jnp.float32),
        pltpu.VMEM((G, HID), jnp.float32),
        pltpu.VMEM((G,), jnp.int32),
        pltpu.VMEM((G,), jnp.int32),
        pltpu.VMEM((16,), jnp.float32),
        pltpu.VMEM_SHARED((NPAD, HID), jnp.float32),
        pltpu.SemaphoreType.DMA,
        pltpu.SemaphoreType.DMA,
        pltpu.SemaphoreType.DMA,
        pltpu.SemaphoreType.DMA,
        pltpu.SemaphoreType.DMA,
        pltpu.SemaphoreType.DMA,
    ],
)()


def _softmax_aggregate(v, pk, zeros, t):
    u = v + 1e-7
    u2 = jnp.concatenate([u[:, :HALF], u[:, HALF:]], axis=0)
    tvec = jnp.full((16,), t, jnp.float32)
    out = _genconv_sc(u2, pk, zeros, tvec)
    return jnp.concatenate([out[0, :N], out[1, :N]], axis=1)


def _pack_edges(src, dst):
    pad = EPAD - E
    src_pad = jnp.concatenate([src, jnp.zeros((pad,), src.dtype)])
    dst_pad = jnp.concatenate([dst, jnp.full((pad,), N, dst.dtype)])
    d = dst_pad.reshape(-1, G)
    pks = [
        jnp.concatenate([(src_pad + c * N).reshape(-1, G), d],
                        axis=1).reshape(-1)
        for c in range(2)
    ]
    return jnp.stack(pks)


def _layernorm(x, g, b):
    mu = jnp.mean(x, axis=-1, keepdims=True)
    var = jnp.var(x, axis=-1, keepdims=True)
    return (x - mu) / jnp.sqrt(var + 1e-5) * g + b


def _genconv_mlp(x, aggr, p):
    h = x + aggr
    h = h @ p['W1'] + p['b1']
    h = _layernorm(h, p['lg'], p['lb'])
    h = jax.nn.relu(h)
    return h @ p['W2'] + p['b2']


def _attn_body(x_ref, wphi_ref, bphi_ref, wa_ref, ba_ref, wb_ref, bb_ref,
               wc_ref, bc_ref, wrho_ref, brho_ref, wcls_ref, bcls_ref,
               logits_ref, a_path_ref, hfin_ref):
    xb = x_ref[0]
    hp = jnp.maximum(
        jnp.dot(xb, wphi_ref[...], preferred_element_type=jnp.float32)
        + bphi_ref[...], 0.0)
    a = jnp.tanh(
        jnp.dot(hp, wa_ref[...], preferred_element_type=jnp.float32)
        + ba_ref[...])
    bg = jax.nn.sigmoid(
        jnp.dot(hp, wb_ref[...], preferred_element_type=jnp.float32)
        + bb_ref[...])
    gated = a * bg
    A = (jnp.dot(gated, wc_ref[...], preferred_element_type=jnp.float32)
         + bc_ref[...])
    a_path_ref[0, 0, :] = A[:, 0]
    m = jnp.max(A, axis=0, keepdims=True)
    e = jnp.exp(A - m)
    sm = e / jnp.sum(e, axis=0, keepdims=True)
    pooled = jnp.dot(sm.T, hp, preferred_element_type=jnp.float32)
    hf = jnp.maximum(
        jnp.dot(pooled, wrho_ref[...], preferred_element_type=jnp.float32)
        + brho_ref[...], 0.0)
    hfin_ref[0] = hf
    logits_ref[0] = (
        jnp.dot(hf, wcls_ref[...], preferred_element_type=jnp.float32)
        + bcls_ref[...])


def _attn_pool(x_, p):
    row = lambda v: v.reshape(1, -1)
    w_spec = lambda s: pl.BlockSpec(s, lambda b: (0,) * len(s))
    out_shapes = (
        jax.ShapeDtypeStruct((B, 1, NC), jnp.float32),
        jax.ShapeDtypeStruct((B, 1, ROWS), jnp.float32),
        jax.ShapeDtypeStruct((B, 1, FEAT), jnp.float32),
    )
    return pl.pallas_call(
        _attn_body,
        grid=(B,),
        out_shape=out_shapes,
        in_specs=[
            pl.BlockSpec((1, ROWS, FEAT), lambda b: (b, 0, 0)),
            w_spec((FEAT, FEAT)), w_spec((1, FEAT)),
            w_spec((FEAT, FEAT)), w_spec((1, FEAT)),
            w_spec((FEAT, FEAT)), w_spec((1, FEAT)),
            w_spec((FEAT, 1)), w_spec((1, 1)),
            w_spec((FEAT, FEAT)), w_spec((1, FEAT)),
            w_spec((FEAT, NC)), w_spec((1, NC)),
        ],
        out_specs=(
            pl.BlockSpec((1, 1, NC), lambda b: (b, 0, 0)),
            pl.BlockSpec((1, 1, ROWS), lambda b: (b, 0, 0)),
            pl.BlockSpec((1, 1, FEAT), lambda b: (b, 0, 0)),
        ),
    )(
        x_.reshape(B, ROWS, FEAT),
        p['Wphi'], row(p['bphi']),
        p['Wa'], row(p['ba']),
        p['Wb'], row(p['bb']),
        p['Wc'], p['bc'].reshape(1, 1),
        p['Wrho'], row(p['brho']),
        p['Wcls'], row(p['bcls']),
    )


def _dsl_body(xc_ref, wg1_ref, bg1_ref, wg2_ref, bg2_ref, out_ref):
    xc = xc_ref[...]
    d2 = jnp.sum(xc * xc, axis=1, keepdims=True)
    nd = 2.0 * jnp.dot(xc, xc.T, preferred_element_type=jnp.float32) \
        - d2 - d2.T
    col = jax.lax.broadcasted_iota(jnp.int32, (BUF, BUF), 1)
    wd = jnp.zeros((BUF, BUF), jnp.float32)
    z = jnp.zeros((BUF, 1), jnp.float32)
    m0 = jnp.max(nd, axis=1, keepdims=True)
    for _ in range(K):
        m = jnp.max(nd, axis=1, keepdims=True)
        eq = nd >= m
        first = jnp.min(jnp.where(eq, col, BUF), axis=1, keepdims=True)
        oh = (col == first).astype(jnp.float32)
        e = jnp.exp(m - m0)
        wd = wd + e * oh
        z = z + e
        nd = jnp.where(oh > 0.0, -3.0e38, nd)
    zinv = 1.0 / z
    hi = jax.lax.Precision.HIGHEST
    agg = jnp.dot(wd, xc, preferred_element_type=jnp.float32,
                  precision=hi) * zinv
    hg = jnp.maximum(
        jnp.dot(agg, wg1_ref[...], preferred_element_type=jnp.float32)
        + bg1_ref[...], 0.0)
    agg2 = jnp.dot(wd, hg, preferred_element_type=jnp.float32,
                   precision=hi) * zinv
    out_ref[...] = (
        jnp.dot(agg2, wg2_ref[...], preferred_element_type=jnp.float32)
        + bg2_ref[...])


def _dsl_graph(x_concat, p):
    return pl.pallas_call(
        _dsl_body,
        out_shape=jax.ShapeDtypeStruct((BUF, NC), jnp.float32),
    )(x_concat, p['Wg1'], p['bg1'].reshape(1, -1),
      p['Wg2'], p['bg2'].reshape(1, -1))


def kernel(x, edge_index, edge_latent, y, params):
    p = params
    pk = _pack_edges(edge_index[0], edge_index[1])
    zeros = jnp.zeros((RPT, HID), jnp.float32)

    h = jax.nn.relu(x @ p['Wfc'] + p['bfc'])
    x_ = h
    g0 = p['gens'][0]
    aggr = _softmax_aggregate(h, pk, zeros, g0['t'])
    cur = _genconv_mlp(h, aggr, g0)
    x_ = jnp.concatenate([x_, cur], axis=-1)
    for g in p['gens'][1:]:
        t = jax.nn.relu(_layernorm(cur, g['ng'], g['nb']))
        aggr = _softmax_aggregate(t, pk, zeros, g['t'])
        cur = cur + _genconv_mlp(t, aggr, g)
        x_ = jnp.concatenate([x_, cur], axis=-1)

    logits3, A_path, h_fin3 = _attn_pool(x_, p)
    logits = logits3.reshape(B, NC)
    h_fin = h_fin3.reshape(B, FEAT)

    x_concat = jnp.concatenate([h_fin, p['rehearsal']], axis=0)[:BUF]
    logits_graph = _dsl_graph(x_concat, p)[:B]
    return logits, A_path, logits_graph
